# baseline (speedup 1.0000x reference)
import jax
import jax.numpy as jnp
import numpy as np
from jax.experimental import pallas as pl
from jax.experimental.pallas import tpu as pltpu

B, S, D = 4, 8192, 768
BLK = 2048
NJ = S // BLK
K = S // 8

_TOP = np.uint32(0x80000000)


def _keys(x_ref, wr_ref):
    wv = jax.lax.dot_general(
        x_ref[0], wr_ref[...], (((1,), (1,)), ((), ())),
        preferred_element_type=jnp.float32)[:, :1]
    u = jax.lax.bitcast_convert_type(wv, jnp.uint32)
    return jnp.where((u & _TOP) != 0, ~u, u | _TOP)


def _mod_kernel(x_ref, wr_ref, W_ref, bb_ref, o_ref, keys_scr, kthr_scr):
    b = pl.program_id(0)
    ph = pl.program_id(1)
    j = pl.program_id(2)

    @pl.when(ph == 0)
    def _scores():
        keys_scr[b, j] = _keys(x_ref, wr_ref).T

        @pl.when(j == NJ - 1)
        def _find_threshold():
            keys = keys_scr[b]

            def body(_, lohi):
                lo, hi = lohi
                span = hi - lo
                mid = lo + (span >> 1) + (span & np.uint32(1))
                cnt = jnp.sum((keys >= mid).astype(jnp.int32), axis=2,
                              keepdims=True)
                cnt = jnp.sum(cnt, axis=0, keepdims=True)
                sel = cnt >= K
                return (jnp.where(sel, mid, lo),
                        jnp.where(sel, hi, mid - np.uint32(1)))

            lo0 = jnp.zeros((1, 1, 1), jnp.uint32)
            hi0 = jnp.full((1, 1, 1), 0xFFFFFFFF, jnp.uint32)
            lo, _ = jax.lax.fori_loop(0, 32, body, (lo0, hi0))
            kthr_scr[b] = jnp.broadcast_to(lo[0], (BLK, 1))

    @pl.when(ph == 1)
    def _output():
        xb = x_ref[0]
        mask = _keys(x_ref, wr_ref) > kthr_scr[b]
        y = jnp.dot(xb.astype(jnp.bfloat16), W_ref[...],
                    preferred_element_type=jnp.float32) + bb_ref[...]
        o_ref[0] = jnp.where(mask, y, xb)


def kernel(x, W_router, b_router, W_block, b_block):
    del b_router
    wr = jnp.zeros((128, D), jnp.float32).at[0].set(W_router[:, 0])
    W16 = W_block.astype(jnp.bfloat16)
    bb = b_block.reshape(1, D)

    out = pl.pallas_call(
        _mod_kernel,
        grid=(B, 2, NJ),
        in_specs=[
            pl.BlockSpec((1, BLK, D), lambda b, ph, j: (b, j, 0)),
            pl.BlockSpec((128, D), lambda b, ph, j: (0, 0)),
            pl.BlockSpec((D, D), lambda b, ph, j: (0, 0)),
            pl.BlockSpec((1, D), lambda b, ph, j: (0, 0)),
        ],
        out_specs=pl.BlockSpec((1, BLK, D),
                               lambda b, ph, j: (b, ph * j, 0)),
        out_shape=jax.ShapeDtypeStruct((B, S, D), jnp.float32),
        scratch_shapes=[
            pltpu.VMEM((B, NJ, 1, BLK), jnp.uint32),
            pltpu.VMEM((B, BLK, 1), jnp.uint32),
        ],
    )(x, wr, W16, bb)
    return out


# speedup vs baseline: 1.1597x; 1.1597x over previous
import jax
import jax.numpy as jnp
import numpy as np
from jax.experimental import pallas as pl
from jax.experimental.pallas import tpu as pltpu

B, S, D = 4, 8192, 768
BLK = 2048
NJ = S // BLK
K = S // 8

_TOP = np.uint32(0x80000000)


def _keys(x_ref, wr_ref):
    wv = jax.lax.dot_general(
        x_ref[0], wr_ref[...], (((1,), (1,)), ((), ())),
        preferred_element_type=jnp.float32)[:, :1]
    u = jax.lax.bitcast_convert_type(wv, jnp.uint32)
    return jnp.where((u & _TOP) != 0, ~u, u | _TOP)


def _mod_kernel(x_ref, wr_ref, W_ref, bb_ref, o_ref, keys_scr, keyp_scr,
                kthr_scr):
    ph = pl.program_id(0)
    b = pl.program_id(1)
    j = pl.program_id(2)

    @pl.when(ph == 0)
    def _scores():
        key = _keys(x_ref, wr_ref)
        keyp_scr[b, j] = key
        keys_scr[b, j] = key.T

        @pl.when((b == B - 1) & (j == NJ - 1))
        def _find_threshold():
            keys = keys_scr[...]

            def body(_, lohi):
                lo, hi = lohi
                span = hi - lo
                mid = lo + (span >> 1) + (span & np.uint32(1))
                cnt = jnp.sum((keys >= mid).astype(jnp.int32), axis=3,
                              keepdims=True)
                cnt = jnp.sum(cnt, axis=1, keepdims=True)
                sel = cnt >= K
                return (jnp.where(sel, mid, lo),
                        jnp.where(sel, hi, mid - np.uint32(1)))

            lo0 = jnp.zeros((B, 1, 1, 1), jnp.uint32)
            hi0 = jnp.full((B, 1, 1, 1), 0xFFFFFFFF, jnp.uint32)
            lo, _ = jax.lax.fori_loop(0, 32, body, (lo0, hi0))
            kthr_scr[...] = jnp.broadcast_to(lo[:, 0], (B, BLK, 1))

    @pl.when(ph == 1)
    def _output():
        xb = x_ref[0]
        mask = keyp_scr[b, j] > kthr_scr[b]
        y = jnp.dot(xb.astype(jnp.bfloat16), W_ref[...],
                    preferred_element_type=jnp.float32) + bb_ref[...]
        o_ref[0] = jnp.where(mask, y, xb)


def kernel(x, W_router, b_router, W_block, b_block):
    del b_router
    wr = jnp.zeros((128, D), jnp.float32).at[0].set(W_router[:, 0])
    W16 = W_block.astype(jnp.bfloat16)
    bb = b_block.reshape(1, D)

    out = pl.pallas_call(
        _mod_kernel,
        grid=(2, B, NJ),
        in_specs=[
            pl.BlockSpec((1, BLK, D), lambda ph, b, j: (b, j, 0)),
            pl.BlockSpec((128, D), lambda ph, b, j: (0, 0)),
            pl.BlockSpec((D, D), lambda ph, b, j: (0, 0)),
            pl.BlockSpec((1, D), lambda ph, b, j: (0, 0)),
        ],
        out_specs=pl.BlockSpec((1, BLK, D),
                               lambda ph, b, j: (ph * b, ph * j, 0)),
        out_shape=jax.ShapeDtypeStruct((B, S, D), jnp.float32),
        scratch_shapes=[
            pltpu.VMEM((B, NJ, 1, BLK), jnp.uint32),
            pltpu.VMEM((B, NJ, BLK, 1), jnp.uint32),
            pltpu.VMEM((B, BLK, 1), jnp.uint32),
        ],
    )(x, wr, W16, bb)
    return out
